# baseline (speedup 1.0000x reference)
import functools

import jax
import jax.numpy as jnp
from jax import lax
from jax.experimental import pallas as pl
from jax.experimental.pallas import tpu as pltpu
from jax.experimental.pallas import tpu_sc as plsc

_R = 16384
_F = 26
_CARD = 64
_W = _F * _CARD
_NW = 32
_RPW = _R // _NW
_CROWS = 32
_NCHUNK = _RPW // _CROWS
_CVALS = _CROWS * _F
_L = 16


def _body(xt_hbm, out_hbm, idx_v, buf0, buf1, rtab, ctab, sem0, sem1):
    wid = lax.axis_index("s") * 2 + lax.axis_index("c")
    row0 = wid * _RPW
    pltpu.sync_copy(xt_hbm.at[:, pl.ds(row0, _RPW)], idx_v)

    iota = lax.iota(jnp.int32, _L)
    ones = jnp.full((_L,), 1, jnp.int32)
    zeros = jnp.zeros((_L,), jnp.int32)
    bufs = (buf0, buf1)
    sems = (sem0, sem1)

    def tinit(i, c):
        g = iota + i * _L
        r = g // _F
        f = g - r * _F
        rtab[pl.ds(i * _L, _L)] = r
        ctab[pl.ds(i * _L, _L)] = f
        return c

    lax.fori_loop(0, _CVALS // _L, tinit, 0)

    def zinit(i, c):
        def zrow(rr, c2):
            buf0[rr, pl.ds(i * _L, _L)] = zeros
            buf1[rr, pl.ds(i * _L, _L)] = zeros
            return c2

        lax.fori_loop(0, _CROWS, zrow, 0)
        return c

    lax.fori_loop(0, _W // _L, zinit, 0)

    def poke(ci, b, val):
        rbase = ci * _CROWS

        def step(i, c):
            r = rtab[pl.ds(i * _L, _L)]
            f = ctab[pl.ds(i * _L, _L)]
            vals = plsc.load_gather(idx_v, [f, rbase + r])
            plsc.store_scatter(bufs[b], [r, f * _CARD + vals], val)
            return c

        lax.fori_loop(0, _CVALS // _L, step, 0)

    def start(ci, b):
        pltpu.make_async_copy(
            bufs[b],
            out_hbm.at[pl.ds(row0 + ci * _CROWS, _CROWS), :],
            sems[b],
        ).start()

    def drain(b):
        pltpu.make_async_copy(
            bufs[b],
            out_hbm.at[pl.ds(row0, _CROWS), :],
            sems[b],
        ).wait()

    for b in (0, 1):
        poke(b, b, ones)
        start(b, b)

    def pair(p, c):
        for b in (0, 1):
            ci = 2 * p + b
            drain(b)
            poke(ci - 2, b, zeros)
            poke(ci, b, ones)
            start(ci, b)
        return c

    lax.fori_loop(1, _NCHUNK // 2, pair, 0)
    drain(0)
    drain(1)


@jax.jit
def _onehot(x):
    mesh = plsc.VectorSubcoreMesh(core_axis_name="c", subcore_axis_name="s")
    f = functools.partial(
        pl.kernel,
        mesh=mesh,
        out_type=jax.ShapeDtypeStruct((_R, _W), jnp.int32),
        scratch_types=[
            pltpu.VMEM((_F, _RPW), jnp.int32),
            pltpu.VMEM((_CROWS, _W), jnp.int32),
            pltpu.VMEM((_CROWS, _W), jnp.int32),
            pltpu.VMEM((_CVALS,), jnp.int32),
            pltpu.VMEM((_CVALS,), jnp.int32),
            pltpu.SemaphoreType.DMA,
            pltpu.SemaphoreType.DMA,
        ],
        compiler_params=pltpu.CompilerParams(needs_layout_passes=False),
    )(_body)
    return f(x)


def kernel(x):
    return _onehot(x.astype(jnp.int32).T)


# speedup vs baseline: 1.1756x; 1.1756x over previous
import functools

import jax
import jax.numpy as jnp
from jax import lax
from jax.experimental import pallas as pl
from jax.experimental.pallas import tpu as pltpu
from jax.experimental.pallas import tpu_sc as plsc

_R = 16384
_F = 26
_CARD = 64
_W = _F * _CARD
_NW = 32
_RPW = _R // _NW
_CROWS = 16
_NCHUNK = _RPW // _CROWS
_CVALS = _CROWS * _F
_L = 16


def _body(x_hbm, out_hbm, idx_v, buf0, buf1, rtab, ctab, sem0, sem1):
    wid = lax.axis_index("s") * 2 + lax.axis_index("c")
    row0 = wid * _RPW
    pltpu.sync_copy(x_hbm.at[pl.ds(row0, _RPW), :], idx_v)

    iota = lax.iota(jnp.int32, _L)
    ones = jnp.full((_L,), 1, jnp.int32)
    zeros = jnp.zeros((_L,), jnp.int32)
    bufs = (buf0, buf1)
    sems = (sem0, sem1)

    def tinit(i, c):
        g = iota + i * _L
        r = g // _F
        f = g - r * _F
        rtab[pl.ds(i * _L, _L)] = r
        ctab[pl.ds(i * _L, _L)] = f
        return c

    lax.fori_loop(0, _CVALS // _L, tinit, 0)

    def zinit(i, c):
        def zrow(rr, c2):
            buf0[rr, pl.ds(i * _L, _L)] = zeros
            buf1[rr, pl.ds(i * _L, _L)] = zeros
            return c2

        lax.fori_loop(0, _CROWS, zrow, 0)
        return c

    lax.fori_loop(0, _W // _L, zinit, 0)

    def poke(ci, b, val):
        rbase = ci * _CROWS

        def step(i, c):
            r = rtab[pl.ds(i * _L, _L)]
            f = ctab[pl.ds(i * _L, _L)]
            vals = plsc.load_gather(idx_v, [rbase + r, f])
            plsc.store_scatter(bufs[b], [r, f * _CARD + vals], val)
            return c

        lax.fori_loop(0, _CVALS // _L, step, 0)

    def start(ci, b):
        pltpu.make_async_copy(
            bufs[b],
            out_hbm.at[pl.ds(row0 + ci * _CROWS, _CROWS), :],
            sems[b],
        ).start()

    def drain(b):
        pltpu.make_async_copy(
            bufs[b],
            out_hbm.at[pl.ds(row0, _CROWS), :],
            sems[b],
        ).wait()

    for b in (0, 1):
        poke(b, b, ones)
        start(b, b)

    def pair(p, c):
        for b in (0, 1):
            ci = 2 * p + b
            drain(b)
            poke(ci - 2, b, zeros)
            poke(ci, b, ones)
            start(ci, b)
        return c

    lax.fori_loop(1, _NCHUNK // 2, pair, 0)
    drain(0)
    drain(1)


@jax.jit
def _onehot(x):
    mesh = plsc.VectorSubcoreMesh(core_axis_name="c", subcore_axis_name="s")
    f = functools.partial(
        pl.kernel,
        mesh=mesh,
        out_type=jax.ShapeDtypeStruct((_R, _W), jnp.int32),
        scratch_types=[
            pltpu.VMEM((_RPW, _F), jnp.int32),
            pltpu.VMEM((_CROWS, _W), jnp.int32),
            pltpu.VMEM((_CROWS, _W), jnp.int32),
            pltpu.VMEM((_CVALS,), jnp.int32),
            pltpu.VMEM((_CVALS,), jnp.int32),
            pltpu.SemaphoreType.DMA,
            pltpu.SemaphoreType.DMA,
        ],
        compiler_params=pltpu.CompilerParams(needs_layout_passes=False),
    )(_body)
    return f(x)


def kernel(x):
    return _onehot(x.astype(jnp.int32))


# speedup vs baseline: 1.2728x; 1.0827x over previous
import functools

import jax
import jax.numpy as jnp
from jax import lax
from jax.experimental import pallas as pl
from jax.experimental.pallas import tpu as pltpu
from jax.experimental.pallas import tpu_sc as plsc

_R = 16384
_F = 26
_CARD = 64
_W = _F * _CARD
_NW = 32
_RPW = _R // _NW
_CROWS = 16
_NCHUNK = _RPW // _CROWS
_CVALS = _CROWS * _F
_L = 16


def _body(x_hbm, out_hbm, idx_v, buf0, buf1, rtab, ctab, sem0, sem1):
    wid = lax.axis_index("s") * 2 + lax.axis_index("c")
    row0 = wid * _RPW
    pltpu.sync_copy(x_hbm.at[pl.ds(row0, _RPW), :], idx_v)

    iota = lax.iota(jnp.int32, _L)
    ones = jnp.full((_L,), 1, jnp.int32)
    zeros = jnp.zeros((_L,), jnp.int32)
    bufs = (buf0, buf1)
    sems = (sem0, sem1)

    def tinit(i, c):
        g = iota + i * _L
        r = g // _F
        f = g - r * _F
        rtab[pl.ds(i * _L, _L)] = r
        ctab[pl.ds(i * _L, _L)] = f
        return c

    lax.fori_loop(0, _CVALS // _L, tinit, 0)

    def zinit(i, c):
        def zrow(rr, c2):
            for k in range(4):
                buf0[rr, pl.ds((i * 4 + k) * _L, _L)] = zeros
                buf1[rr, pl.ds((i * 4 + k) * _L, _L)] = zeros
            return c2

        lax.fori_loop(0, _CROWS, zrow, 0)
        return c

    lax.fori_loop(0, _W // (_L * 4), zinit, 0)

    def poke(ci, b, val):
        rbase = ci * _CROWS

        def step(i, c):
            r = rtab[pl.ds(i * _L, _L)]
            f = ctab[pl.ds(i * _L, _L)]
            vals = plsc.load_gather(idx_v, [rbase + r, f])
            plsc.store_scatter(bufs[b], [r, f * _CARD + vals], val)
            return c

        lax.fori_loop(0, _CVALS // _L, step, 0)

    def start(ci, b):
        pltpu.make_async_copy(
            bufs[b],
            out_hbm.at[pl.ds(row0 + ci * _CROWS, _CROWS), :],
            sems[b],
        ).start()

    def drain(b):
        pltpu.make_async_copy(
            bufs[b],
            out_hbm.at[pl.ds(row0, _CROWS), :],
            sems[b],
        ).wait()

    for b in (0, 1):
        poke(b, b, ones)
        start(b, b)

    def pair(p, c):
        for b in (0, 1):
            ci = 2 * p + b
            drain(b)
            poke(ci - 2, b, zeros)
            poke(ci, b, ones)
            start(ci, b)
        return c

    lax.fori_loop(1, _NCHUNK // 2, pair, 0)
    drain(0)
    drain(1)


@jax.jit
def _onehot(x):
    mesh = plsc.VectorSubcoreMesh(core_axis_name="c", subcore_axis_name="s")
    f = functools.partial(
        pl.kernel,
        mesh=mesh,
        out_type=jax.ShapeDtypeStruct((_R, _W), jnp.int32),
        scratch_types=[
            pltpu.VMEM((_RPW, _F), jnp.int32),
            pltpu.VMEM((_CROWS, _W), jnp.int32),
            pltpu.VMEM((_CROWS, _W), jnp.int32),
            pltpu.VMEM((_CVALS,), jnp.int32),
            pltpu.VMEM((_CVALS,), jnp.int32),
            pltpu.SemaphoreType.DMA,
            pltpu.SemaphoreType.DMA,
        ],
        compiler_params=pltpu.CompilerParams(needs_layout_passes=False),
    )(_body)
    return f(x)


def kernel(x):
    return _onehot(x.astype(jnp.int32))


# speedup vs baseline: 1.3127x; 1.0314x over previous
import functools

import jax
import jax.numpy as jnp
from jax import lax
from jax.experimental import pallas as pl
from jax.experimental.pallas import tpu as pltpu
from jax.experimental.pallas import tpu_sc as plsc

_R = 16384
_F = 26
_CARD = 64
_W = _F * _CARD
_NW = 32
_RPW = _R // _NW
_CROWS = 16
_NCHUNK = _RPW // _CROWS
_CVALS = _CROWS * _F
_L = 16


def _body(x_hbm, out_hbm, idx_v, buf0, buf1, rtab, ctab, sem0, sem1):
    wid = lax.axis_index("s") * 2 + lax.axis_index("c")
    row0 = wid * _RPW
    idx_cp = pltpu.make_async_copy(x_hbm.at[pl.ds(row0, _RPW), :], idx_v, sem0)
    idx_cp.start()

    iota = lax.iota(jnp.int32, _L)
    ones = jnp.full((_L,), 1, jnp.int32)
    zeros = jnp.zeros((_L,), jnp.int32)
    bufs = (buf0, buf1)
    sems = (sem0, sem1)

    def tinit(i, c):
        g = iota + i * _L
        r = g // _F
        f = g - r * _F
        rtab[pl.ds(i * _L, _L)] = r
        ctab[pl.ds(i * _L, _L)] = f
        return c

    lax.fori_loop(0, _CVALS // _L, tinit, 0)

    def zinit(i, c):
        def zrow(rr, c2):
            for k in range(4):
                buf0[rr, pl.ds((i * 4 + k) * _L, _L)] = zeros
                buf1[rr, pl.ds((i * 4 + k) * _L, _L)] = zeros
            return c2

        lax.fori_loop(0, _CROWS, zrow, 0)
        return c

    lax.fori_loop(0, _W // (_L * 4), zinit, 0)
    idx_cp.wait()

    def poke(ci, b, val):
        rbase = ci * _CROWS

        def step(i, c):
            for k in (0, 1):
                j = 2 * i + k
                r = rtab[pl.ds(j * _L, _L)]
                f = ctab[pl.ds(j * _L, _L)]
                vals = plsc.load_gather(idx_v, [rbase + r, f])
                plsc.store_scatter(bufs[b], [r, f * _CARD + vals], val)
            return c

        lax.fori_loop(0, _CVALS // (2 * _L), step, 0)

    def start(ci, b):
        pltpu.make_async_copy(
            bufs[b],
            out_hbm.at[pl.ds(row0 + ci * _CROWS, _CROWS), :],
            sems[b],
        ).start()

    def drain(b):
        pltpu.make_async_copy(
            bufs[b],
            out_hbm.at[pl.ds(row0, _CROWS), :],
            sems[b],
        ).wait()

    for b in (0, 1):
        poke(b, b, ones)
        start(b, b)

    def pair(p, c):
        for b in (0, 1):
            ci = 2 * p + b
            drain(b)
            poke(ci - 2, b, zeros)
            poke(ci, b, ones)
            start(ci, b)
        return c

    lax.fori_loop(1, _NCHUNK // 2, pair, 0)
    drain(0)
    drain(1)


@jax.jit
def _onehot(x):
    mesh = plsc.VectorSubcoreMesh(core_axis_name="c", subcore_axis_name="s")
    f = functools.partial(
        pl.kernel,
        mesh=mesh,
        out_type=jax.ShapeDtypeStruct((_R, _W), jnp.int32),
        scratch_types=[
            pltpu.VMEM((_RPW, _F), jnp.int32),
            pltpu.VMEM((_CROWS, _W), jnp.int32),
            pltpu.VMEM((_CROWS, _W), jnp.int32),
            pltpu.VMEM((_CVALS,), jnp.int32),
            pltpu.VMEM((_CVALS,), jnp.int32),
            pltpu.SemaphoreType.DMA,
            pltpu.SemaphoreType.DMA,
        ],
        compiler_params=pltpu.CompilerParams(needs_layout_passes=False),
    )(_body)
    return f(x)


def kernel(x):
    return _onehot(x.astype(jnp.int32))
